# Initial kernel scaffold; baseline (speedup 1.0000x reference)
import jax
import jax.numpy as jnp
from jax.experimental import pallas as pl


def kernel(cat_features, emb_table, lin_table, bias):
    raise NotImplementedError("write your pallas kernel here")


# speedup vs baseline: 7.5063x; 7.5063x over previous
import functools

import jax
import jax.numpy as jnp
from jax import lax
from jax.experimental import pallas as pl
from jax.experimental.pallas import tpu as pltpu
from jax.experimental.pallas import tpu_sc as plsc

_NC = 2
_NS = 16
_L = 16
_NBUF = 4


def _fm_body(F, FP, K, BPW, cat_hbm, emb_hbm, lin_hbm, bias_hbm, out_hbm,
             idx_v, out_v, bias_v, *bufs):
  ebufs = bufs[:_NBUF]
  lbufs = bufs[_NBUF:2 * _NBUF]
  esems = bufs[2 * _NBUF:3 * _NBUF]
  lsems = bufs[3 * _NBUF:4 * _NBUF]
  KG = K // _L

  wid = lax.axis_index("s") * _NC + lax.axis_index("c")
  base = wid * BPW

  pltpu.sync_copy(cat_hbm.at[pl.ds(base, BPW)], idx_v)
  pltpu.sync_copy(bias_hbm, bias_v)
  bias0 = bias_v[...][0]

  def _fire(j, b):
    pltpu.async_copy(emb_hbm.at[idx_v.at[j]], ebufs[b], esems[b])
    pltpu.async_copy(lin_hbm.at[idx_v.at[j]], lbufs[b], lsems[b])

  for b in range(_NBUF):
    _fire(b, b)

  zero = jnp.zeros((_L,), jnp.float32)
  lane_iota = lax.iota(jnp.int32, _L)

  def _gstep(g, resvec):
    for b in range(_NBUF):
      j = g * _NBUF + b
      pltpu.make_async_copy(emb_hbm.at[idx_v.at[j]], ebufs[b], esems[b]).wait()
      pltpu.make_async_copy(lin_hbm.at[idx_v.at[j]], lbufs[b], lsems[b]).wait()

      def _accum(f, c):
        ss = c[:KG]
        qq = c[KG:2 * KG]
        ls = c[2 * KG]
        new_ss = []
        new_qq = []
        for gg in range(KG):
          e = ebufs[b][f, pl.ds(gg * _L, _L)]
          new_ss.append(ss[gg] + e)
          new_qq.append(qq[gg] + e * e)
        return (*new_ss, *new_qq, ls + lbufs[b][f])

      res = lax.fori_loop(0, F, _accum, (zero,) * (2 * KG + 1))
      ss = res[:KG]
      qq = res[KG:2 * KG]
      ls = res[2 * KG]
      r = zero
      for gg in range(KG):
        r = r + (ss[gg] * ss[gg] - qq[gg])
      v = 0.5 * r + ls
      total = bias0
      for lane_i in range(_L):
        total = total + v[lane_i]
      lane = lax.rem(j, _L)
      resvec = jnp.where(lane_iota == lane, total, resvec)
      if b == _NBUF - 1:

        @pl.when(lane == _L - 1)
        def _():
          out_v[pl.ds(j - (_L - 1), _L)] = resvec

      nj = j + _NBUF

      @pl.when(nj < BPW)
      def _():
        _fire(nj, b)

    return resvec

  lax.fori_loop(0, BPW // _NBUF, _gstep, zero)
  pltpu.sync_copy(out_v, out_hbm.at[pl.ds(base, BPW)])


def kernel(cat_features, emb_table, lin_table, bias):
  B, F = cat_features.shape
  V, K = emb_table.shape
  NW = _NC * _NS
  BPW = B // NW
  FP = -(-F // 8) * 8

  cat_pad = jnp.pad(cat_features, ((0, 0), (0, FP - F)))
  lin_pad = jnp.pad(lin_table, ((0, 0), (0, _L - lin_table.shape[1])))
  bias_pad = jnp.pad(bias, (0, _L - bias.shape[0]))

  mesh = plsc.VectorSubcoreMesh(core_axis_name="c", subcore_axis_name="s")
  scratch = [
      pltpu.VMEM((BPW, FP), jnp.int32),
      pltpu.VMEM((BPW,), jnp.float32),
      pltpu.VMEM((_L,), jnp.float32),
  ]
  scratch += [pltpu.VMEM((FP, K), jnp.float32) for _ in range(_NBUF)]
  scratch += [pltpu.VMEM((FP, _L), jnp.float32) for _ in range(_NBUF)]
  scratch += [pltpu.SemaphoreType.DMA for _ in range(2 * _NBUF)]

  body = functools.partial(_fm_body, F, FP, K, BPW)
  out = pl.kernel(
      body,
      out_type=jax.ShapeDtypeStruct((B,), jnp.float32),
      mesh=mesh,
      scratch_types=scratch,
      compiler_params=pltpu.CompilerParams(use_tc_tiling_on_sc=False),
  )(cat_pad, emb_table, lin_pad, bias_pad)
  return out.reshape(B, 1)
